# baseline (speedup 1.0000x reference)
import jax
import jax.numpy as jnp
from jax.experimental import pallas as pl
from jax.experimental.pallas import tpu as pltpu

_EPS = 1e-12


def _tc_body(z_ref, w_ref, cb_ref, zq_ref, idx_ref, commit_ref, cn_ref,
             cbb_ref):
    i = pl.program_id(0)
    tn = z_ref.shape[0]
    M = 4
    K = 1024
    Ds = 64

    @pl.when(i == 0)
    def _():
        cb = cb_ref[...]
        cn_ref[...] = cb / jnp.maximum(
            jnp.sqrt(jnp.sum(cb * cb, axis=-1, keepdims=True)), _EPS)
        cbb_ref[...] = cb.astype(jnp.bfloat16)

    z = z_ref[...]
    w = w_ref[...]
    zr = jnp.dot(z, w, preferred_element_type=jnp.float32)

    zz = zr * zr
    idx_cols = []
    q_parts = []
    for m in range(M):
        zs = zr[:, m * Ds:(m + 1) * Ds]
        zn = zs / jnp.maximum(
            jnp.sqrt(jnp.sum(zz[:, m * Ds:(m + 1) * Ds], axis=-1,
                             keepdims=True)), _EPS)
        cn = cn_ref[m * K:(m + 1) * K, :]
        sim = jax.lax.dot_general(
            zn, cn, (((1,), (1,)), ((), ())),
            preferred_element_type=jnp.float32)
        idx = jnp.argmin(1.0 - sim, axis=-1).astype(jnp.int32)
        oh = jnp.where(
            jax.lax.broadcasted_iota(jnp.int16, (tn, K), 1)
            == idx.astype(jnp.int16)[:, None],
            jnp.bfloat16(1), jnp.bfloat16(0))
        cm = cbb_ref[m * K:(m + 1) * K, :]
        qm = jnp.dot(oh, cm, preferred_element_type=jnp.float32)
        idx_cols.append(idx[:, None])
        q_parts.append(qm)

    zq_rot = jnp.concatenate(q_parts, axis=1)
    idx_ref[...] = jnp.concatenate(idx_cols, axis=1)

    st = zr + (zq_rot - zr)
    zq_ref[...] = jax.lax.dot_general(
        st, w, (((1,), (1,)), ((), ())),
        preferred_element_type=jnp.float32)

    diff = zr - zq_rot
    s = jnp.sum(diff * diff)

    @pl.when(i == 0)
    def _():
        commit_ref[0, 0] = s

    @pl.when(i > 0)
    def _():
        commit_ref[0, 0] += s


def kernel(z, W, codebooks):
    B, T, D = z.shape
    M, K, Ds = codebooks.shape
    N = B * T
    TN = 1024
    grid = N // TN

    z_flat = z.reshape(N, D)
    cb_flat = codebooks.reshape(M * K, Ds)

    zq, idx, commit = pl.pallas_call(
        _tc_body,
        grid=(grid,),
        in_specs=[
            pl.BlockSpec((TN, D), lambda i: (i, 0)),
            pl.BlockSpec((D, D), lambda i: (0, 0)),
            pl.BlockSpec((M * K, Ds), lambda i: (0, 0)),
        ],
        out_specs=[
            pl.BlockSpec((TN, D), lambda i: (i, 0)),
            pl.BlockSpec((TN, M), lambda i: (i, 0)),
            pl.BlockSpec((1, 1), lambda i: (0, 0), memory_space=pltpu.SMEM),
        ],
        out_shape=[
            jax.ShapeDtypeStruct((N, D), jnp.float32),
            jax.ShapeDtypeStruct((N, M), jnp.int32),
            jax.ShapeDtypeStruct((1, 1), jnp.float32),
        ],
        scratch_shapes=[pltpu.VMEM((M * K, Ds), jnp.float32),
                        pltpu.VMEM((M * K, Ds), jnp.bfloat16)],
        compiler_params=pltpu.CompilerParams(
            dimension_semantics=("arbitrary",)),
    )(z_flat, W, cb_flat)

    return (zq.reshape(B, T, D), idx.reshape(B, T, M),
            commit[0, 0] / jnp.float32(N * D))


# speedup vs baseline: 1.2438x; 1.2438x over previous
import jax
import jax.numpy as jnp
from jax.experimental import pallas as pl
from jax.experimental.pallas import tpu as pltpu

_EPS = 1e-12


def _tc_body(z_ref, w_ref, cb_ref, zq_ref, idx_ref, commit_ref, cn_ref):
    i = pl.program_id(0)
    tn = z_ref.shape[0]
    M = 4
    K = 1024
    Ds = 64

    @pl.when(i == 0)
    def _():
        cb = cb_ref[...]
        cn_ref[...] = cb / jnp.maximum(
            jnp.sqrt(jnp.sum(cb * cb, axis=-1, keepdims=True)), _EPS)

    z = z_ref[...]
    w = w_ref[...]
    zr = jnp.dot(z, w, preferred_element_type=jnp.float32)

    zz = zr * zr
    idx_cols = []
    q_parts = []
    for m in range(M):
        zs = zr[:, m * Ds:(m + 1) * Ds]
        zn = zs / jnp.maximum(
            jnp.sqrt(jnp.sum(zz[:, m * Ds:(m + 1) * Ds], axis=-1,
                             keepdims=True)), _EPS)
        cn = cn_ref[m * K:(m + 1) * K, :]
        sim = jax.lax.dot_general(
            zn, cn, (((1,), (1,)), ((), ())),
            preferred_element_type=jnp.float32)
        idx = jnp.argmin(1.0 - sim, axis=-1).astype(jnp.int32)
        oh = (jax.lax.broadcasted_iota(jnp.int32, (tn, K), 1)
              == idx[:, None]).astype(jnp.float32)
        cm = cb_ref[m * K:(m + 1) * K, :]
        qm = jnp.dot(oh, cm, preferred_element_type=jnp.float32)
        idx_cols.append(idx[:, None])
        q_parts.append(qm)

    zq_rot = jnp.concatenate(q_parts, axis=1)
    idx_ref[...] = jnp.concatenate(idx_cols, axis=1)

    st = zr + (zq_rot - zr)
    zq_ref[...] = jax.lax.dot_general(
        st, w, (((1,), (1,)), ((), ())),
        preferred_element_type=jnp.float32)

    diff = zr - zq_rot
    s = jnp.sum(diff * diff)

    @pl.when(i == 0)
    def _():
        commit_ref[0, 0] = s

    @pl.when(i > 0)
    def _():
        commit_ref[0, 0] += s


def kernel(z, W, codebooks):
    B, T, D = z.shape
    M, K, Ds = codebooks.shape
    N = B * T
    TN = 2048
    grid = N // TN

    z_flat = z.reshape(N, D)
    cb_flat = codebooks.reshape(M * K, Ds)

    zq, idx, commit = pl.pallas_call(
        _tc_body,
        grid=(grid,),
        in_specs=[
            pl.BlockSpec((TN, D), lambda i: (i, 0)),
            pl.BlockSpec((D, D), lambda i: (0, 0)),
            pl.BlockSpec((M * K, Ds), lambda i: (0, 0)),
        ],
        out_specs=[
            pl.BlockSpec((TN, D), lambda i: (i, 0)),
            pl.BlockSpec((TN, M), lambda i: (i, 0)),
            pl.BlockSpec((1, 1), lambda i: (0, 0), memory_space=pltpu.SMEM),
        ],
        out_shape=[
            jax.ShapeDtypeStruct((N, D), jnp.float32),
            jax.ShapeDtypeStruct((N, M), jnp.int32),
            jax.ShapeDtypeStruct((1, 1), jnp.float32),
        ],
        scratch_shapes=[pltpu.VMEM((M * K, Ds), jnp.float32)],
        compiler_params=pltpu.CompilerParams(
            dimension_semantics=("arbitrary",)),
    )(z_flat, W, cb_flat)

    return (zq.reshape(B, T, D), idx.reshape(B, T, M),
            commit[0, 0] / jnp.float32(N * D))


# speedup vs baseline: 1.2605x; 1.0134x over previous
import jax
import jax.numpy as jnp
from jax.experimental import pallas as pl
from jax.experimental.pallas import tpu as pltpu

_EPS = 1e-12


def _tc_body(z_ref, w_ref, cb_ref, zq_ref, idx_ref, commit_ref, cn_ref):
    i = pl.program_id(0)
    tn = z_ref.shape[0]
    M = 4
    K = 1024
    Ds = 64

    @pl.when(i == 0)
    def _():
        cb = cb_ref[...]
        cn_ref[...] = cb / jnp.maximum(
            jnp.sqrt(jnp.sum(cb * cb, axis=-1, keepdims=True)), _EPS)

    z = z_ref[...]
    w = w_ref[...]
    zr = jnp.dot(z, w, preferred_element_type=jnp.float32)

    zz = zr * zr
    idx_cols = []
    q_parts = []
    for m in range(M):
        zs = zr[:, m * Ds:(m + 1) * Ds]
        zn = zs / jnp.maximum(
            jnp.sqrt(jnp.sum(zz[:, m * Ds:(m + 1) * Ds], axis=-1,
                             keepdims=True)), _EPS)
        cn = cn_ref[m * K:(m + 1) * K, :]
        sim = jax.lax.dot_general(
            zn, cn, (((1,), (1,)), ((), ())),
            preferred_element_type=jnp.float32)
        idx = jnp.argmin(1.0 - sim, axis=-1).astype(jnp.int32)
        oh = (jax.lax.broadcasted_iota(jnp.int32, (tn, K), 1)
              == idx[:, None]).astype(jnp.float32)
        cm = cb_ref[m * K:(m + 1) * K, :]
        qm = jnp.dot(oh, cm, preferred_element_type=jnp.float32)
        idx_cols.append(idx[:, None])
        q_parts.append(qm)

    zq_rot = jnp.concatenate(q_parts, axis=1)
    idx_ref[...] = jnp.concatenate(idx_cols, axis=1)

    st = zr + (zq_rot - zr)
    zq_ref[...] = jax.lax.dot_general(
        st, w, (((1,), (1,)), ((), ())),
        preferred_element_type=jnp.float32)

    diff = zr - zq_rot
    s = jnp.sum(diff * diff)

    @pl.when(i == 0)
    def _():
        commit_ref[0, 0] = s

    @pl.when(i > 0)
    def _():
        commit_ref[0, 0] += s


def kernel(z, W, codebooks):
    B, T, D = z.shape
    M, K, Ds = codebooks.shape
    N = B * T
    TN = 4096
    grid = N // TN

    z_flat = z.reshape(N, D)
    cb_flat = codebooks.reshape(M * K, Ds)

    zq, idx, commit = pl.pallas_call(
        _tc_body,
        grid=(grid,),
        in_specs=[
            pl.BlockSpec((TN, D), lambda i: (i, 0)),
            pl.BlockSpec((D, D), lambda i: (0, 0)),
            pl.BlockSpec((M * K, Ds), lambda i: (0, 0)),
        ],
        out_specs=[
            pl.BlockSpec((TN, D), lambda i: (i, 0)),
            pl.BlockSpec((TN, M), lambda i: (i, 0)),
            pl.BlockSpec((1, 1), lambda i: (0, 0), memory_space=pltpu.SMEM),
        ],
        out_shape=[
            jax.ShapeDtypeStruct((N, D), jnp.float32),
            jax.ShapeDtypeStruct((N, M), jnp.int32),
            jax.ShapeDtypeStruct((1, 1), jnp.float32),
        ],
        scratch_shapes=[pltpu.VMEM((M * K, Ds), jnp.float32)],
        compiler_params=pltpu.CompilerParams(
            dimension_semantics=("arbitrary",)),
    )(z_flat, W, cb_flat)

    return (zq.reshape(B, T, D), idx.reshape(B, T, M),
            commit[0, 0] / jnp.float32(N * D))
